# Initial kernel scaffold; baseline (speedup 1.0000x reference)
import jax
import jax.numpy as jnp
from jax.experimental import pallas as pl


def kernel(expr, jaw_quat_weight, xyz, feat_lines_x, feat_lines_y, feat_lines_z, v0, g0, b0, v1, g1, b1, v2, g2, b2):
    raise NotImplementedError("write your pallas kernel here")


# speedup vs baseline: 29.4365x; 29.4365x over previous
import functools

import jax
import jax.numpy as jnp
from jax.experimental import pallas as pl
from jax.experimental.pallas import tpu as pltpu

EXPR_NUM = 80
KEY_JAW = 16
L = 64
C = 32
N_HIDDEN = 128
N_PTS = 131072

BLOCK = 2048


def _fused_kernel(xyz_ref, u_bs_ref, u_jw_ref, flx_ref, fly_ref, flz_ref,
                  v0_ref, g0_ref, b0_ref, v1_ref, g1_ref, b1_ref,
                  v2_ref, g2_ref, b2_ref,
                  out_ref,
                  mx_ref, my_ref, mz_ref, w1_ref):
    @pl.when(pl.program_id(0) == 0)
    def _prologue():
        v0 = v0_ref[...]
        inv0 = g0_ref[...] * jax.lax.rsqrt(
            jnp.sum(v0 * v0, axis=1, keepdims=True))
        w0 = v0 * inv0
        u_bs = u_bs_ref[...][:, :, None]
        u_jw = u_jw_ref[...][:, :, None]
        for a, fl_ref, m_ref in ((0, flx_ref, mx_ref), (1, fly_ref, my_ref),
                                 (2, flz_ref, mz_ref)):
            fl = fl_ref[...]
            a_bs = jnp.sum(fl * u_bs, axis=0)
            a_jw = jnp.sum(fl * u_jw, axis=0)
            w0_bs = w0[:, 32 * a:32 * (a + 1)]
            w0_jw = w0[:, 96 + 32 * a:96 + 32 * (a + 1)]
            m_ref[...] = (
                jax.lax.dot_general(a_bs, w0_bs, (((1,), (1,)), ((), ())),
                                    preferred_element_type=jnp.float32)
                + jax.lax.dot_general(a_jw, w0_jw, (((1,), (1,)), ((), ())),
                                      preferred_element_type=jnp.float32))
        v1 = v1_ref[...]
        inv1 = g1_ref[...] * jax.lax.rsqrt(
            jnp.sum(v1 * v1, axis=1, keepdims=True))
        w1_ref[...] = v1 * inv1

    xyz = xyz_ref[...]
    p = jnp.clip(xyz, 0.0, 1.0) * (L - 1)
    grid = jax.lax.broadcasted_iota(jnp.int32, (1, L), 1).astype(jnp.float32)

    h = b0_ref[...]
    for a, m_ref in ((0, mx_ref), (1, my_ref), (2, mz_ref)):
        pa = p[:, a:a + 1]
        tent = jnp.maximum(1.0 - jnp.abs(pa - grid), 0.0)
        h = h + jnp.dot(tent, m_ref[...],
                        preferred_element_type=jnp.float32)
    h = jnp.maximum(h, 0.0)

    h = jax.lax.dot_general(h, w1_ref[...], (((1,), (1,)), ((), ())),
                            preferred_element_type=jnp.float32)
    h = jnp.maximum(h + b1_ref[...], 0.0)

    v2 = v2_ref[...]
    w2 = v2 * (g2_ref[...] * jax.lax.rsqrt(jnp.sum(v2 * v2)))
    out_ref[...] = (jnp.sum(h * w2, axis=1, keepdims=True)
                    + b2_ref[...])


@jax.jit
def kernel(expr, jaw_quat_weight, xyz, feat_lines_x, feat_lines_y,
           feat_lines_z, v0, g0, b0, v1, g1, b1, v2, g2, b2):
    e = expr.reshape(-1)[:EXPR_NUM]
    jw = jaw_quat_weight.reshape(-1)
    nb = EXPR_NUM + KEY_JAW
    u_bs = jnp.concatenate([e, jnp.zeros((KEY_JAW,), e.dtype)])[:, None]
    u_jw = jnp.concatenate([jnp.zeros((EXPR_NUM,), jw.dtype), jw])[:, None]

    n = xyz.shape[0]
    grid = (n // BLOCK,)
    const = lambda shape: pl.BlockSpec(shape, lambda i: (0,) * len(shape))

    out = pl.pallas_call(
        _fused_kernel,
        grid=grid,
        in_specs=[
            pl.BlockSpec((BLOCK, 3), lambda i: (i, 0)),
            const((nb, 1)), const((nb, 1)),
            const((nb, L, C)), const((nb, L, C)), const((nb, L, C)),
            const((N_HIDDEN, 6 * C)), const((N_HIDDEN, 1)), const((1, N_HIDDEN)),
            const((N_HIDDEN, N_HIDDEN)), const((N_HIDDEN, 1)), const((1, N_HIDDEN)),
            const((1, N_HIDDEN)), const((1, 1)), const((1, 1)),
        ],
        out_specs=pl.BlockSpec((BLOCK, 1), lambda i: (i, 0)),
        out_shape=jax.ShapeDtypeStruct((n, 1), jnp.float32),
        scratch_shapes=[
            pltpu.VMEM((L, N_HIDDEN), jnp.float32),
            pltpu.VMEM((L, N_HIDDEN), jnp.float32),
            pltpu.VMEM((L, N_HIDDEN), jnp.float32),
            pltpu.VMEM((N_HIDDEN, N_HIDDEN), jnp.float32),
        ],
    )(xyz, u_bs, u_jw, feat_lines_x, feat_lines_y, feat_lines_z,
      v0, g0[:, None], b0[None, :], v1, g1[:, None], b1[None, :],
      v2, g2[:, None], b2[None, :])
    return out


# speedup vs baseline: 84.3555x; 2.8657x over previous
import jax
import jax.numpy as jnp
from jax.experimental import pallas as pl
from jax.experimental.pallas import tpu as pltpu

EXPR_NUM = 80
KEY_JAW = 16
L = 64
C = 32
N_HIDDEN = 128

BLOCK = 4096


def _fused_kernel(xyzt_ref, u_bs_ref, u_jw_ref, flx_ref, fly_ref, flz_ref,
                  v0_ref, g0_ref, b0_ref, v1_ref, g1_ref, b1_ref,
                  v2_ref, g2_ref, b2_ref,
                  out_ref,
                  mcat_ref, w1_ref):
    @pl.when(pl.program_id(0) == 0)
    def _prologue():
        v0 = v0_ref[...]
        inv0 = g0_ref[...] * jax.lax.rsqrt(
            jnp.sum(v0 * v0, axis=1, keepdims=True))
        w0 = v0 * inv0
        u_bs = u_bs_ref[...][:, :, None]
        u_jw = u_jw_ref[...][:, :, None]
        for a, fl_ref in ((0, flx_ref), (1, fly_ref), (2, flz_ref)):
            fl = fl_ref[...]
            a_bs = jnp.sum(fl * u_bs, axis=0)
            a_jw = jnp.sum(fl * u_jw, axis=0)
            w0_bs = w0[:, 32 * a:32 * (a + 1)]
            w0_jw = w0[:, 96 + 32 * a:96 + 32 * (a + 1)]
            mcat_ref[:, 64 * a:64 * (a + 1)] = (
                jnp.dot(w0_bs, a_bs, preferred_element_type=jnp.float32)
                + jnp.dot(w0_jw, a_jw, preferred_element_type=jnp.float32))
        v1 = v1_ref[...]
        inv1 = g1_ref[...] * jax.lax.rsqrt(
            jnp.sum(v1 * v1, axis=1, keepdims=True))
        w1_ref[...] = v1 * inv1

    p = jnp.clip(xyzt_ref[...], 0.0, 1.0) * (L - 1)
    s = jax.lax.broadcasted_iota(jnp.int32, (3 * L, 1), 0)
    offs = (s % L).astype(jnp.float32)
    psel = jnp.where(s < L, p[0:1, :],
                     jnp.where(s < 2 * L, p[1:2, :], p[2:3, :]))
    tent = jnp.maximum(1.0 - jnp.abs(psel - offs), 0.0)

    h = jnp.dot(mcat_ref[...], tent, preferred_element_type=jnp.float32)
    h = jnp.maximum(h + b0_ref[...], 0.0)
    h = jnp.dot(w1_ref[...], h, preferred_element_type=jnp.float32)
    h = jnp.maximum(h + b1_ref[...], 0.0)

    v2 = v2_ref[...]
    w2 = v2 * (g2_ref[...] * jax.lax.rsqrt(jnp.sum(v2 * v2)))
    out_ref[...] = (jnp.sum(h * w2, axis=0, keepdims=True)
                    + b2_ref[...])


@jax.jit
def kernel(expr, jaw_quat_weight, xyz, feat_lines_x, feat_lines_y,
           feat_lines_z, v0, g0, b0, v1, g1, b1, v2, g2, b2):
    e = expr.reshape(-1)[:EXPR_NUM]
    jw = jaw_quat_weight.reshape(-1)
    nb = EXPR_NUM + KEY_JAW
    u_bs = jnp.concatenate([e, jnp.zeros((KEY_JAW,), e.dtype)])[:, None]
    u_jw = jnp.concatenate([jnp.zeros((EXPR_NUM,), jw.dtype), jw])[:, None]

    n = xyz.shape[0]
    xyzt = xyz.T
    flxt = jnp.swapaxes(feat_lines_x, 1, 2)
    flyt = jnp.swapaxes(feat_lines_y, 1, 2)
    flzt = jnp.swapaxes(feat_lines_z, 1, 2)

    grid = (n // BLOCK,)
    const = lambda shape: pl.BlockSpec(shape, lambda i: (0,) * len(shape))

    out = pl.pallas_call(
        _fused_kernel,
        grid=grid,
        in_specs=[
            pl.BlockSpec((3, BLOCK), lambda i: (0, i)),
            const((nb, 1)), const((nb, 1)),
            const((nb, C, L)), const((nb, C, L)), const((nb, C, L)),
            const((N_HIDDEN, 6 * C)), const((N_HIDDEN, 1)), const((N_HIDDEN, 1)),
            const((N_HIDDEN, N_HIDDEN)), const((N_HIDDEN, 1)), const((N_HIDDEN, 1)),
            const((N_HIDDEN, 1)), const((1, 1)), const((1, 1)),
        ],
        out_specs=pl.BlockSpec((1, BLOCK), lambda i: (0, i)),
        out_shape=jax.ShapeDtypeStruct((1, n), jnp.float32),
        scratch_shapes=[
            pltpu.VMEM((N_HIDDEN, 3 * L), jnp.float32),
            pltpu.VMEM((N_HIDDEN, N_HIDDEN), jnp.float32),
        ],
    )(xyzt, u_bs, u_jw, flxt, flyt, flzt,
      v0, g0[:, None], b0[:, None], v1, g1[:, None], b1[:, None],
      v2.T, g2[:, None], b2[None, :])
    return out.reshape(n, 1)


# speedup vs baseline: 84.5734x; 1.0026x over previous
import jax
import jax.numpy as jnp
from jax.experimental import pallas as pl
from jax.experimental.pallas import tpu as pltpu

EXPR_NUM = 80
KEY_JAW = 16
L = 64
C = 32
N_HIDDEN = 128

BLOCK = 4096


def _fused_kernel(xyzt_ref, u_bs_ref, u_jw_ref, flx_ref, fly_ref, flz_ref,
                  v0_ref, g0_ref, b0_ref, v1_ref, g1_ref, b1_ref,
                  v2_ref, g2_ref, b2_ref,
                  out_ref,
                  mcat_ref, w1_ref):
    @pl.when(pl.program_id(0) == 0)
    def _prologue():
        v0 = v0_ref[...]
        inv0 = g0_ref[...] * jax.lax.rsqrt(
            jnp.sum(v0 * v0, axis=1, keepdims=True))
        w0 = v0 * inv0
        u_bs = u_bs_ref[...][:, :, None]
        u_jw = u_jw_ref[...][:, :, None]
        for a, fl_ref in ((0, flx_ref), (1, fly_ref), (2, flz_ref)):
            fl = fl_ref[...]
            a_bs = jnp.sum(fl * u_bs, axis=0)
            a_jw = jnp.sum(fl * u_jw, axis=0)
            w0_bs = w0[:, 32 * a:32 * (a + 1)]
            w0_jw = w0[:, 96 + 32 * a:96 + 32 * (a + 1)]
            mcat_ref[:, 64 * a:64 * (a + 1)] = (
                jnp.dot(w0_bs, a_bs, preferred_element_type=jnp.float32)
                + jnp.dot(w0_jw, a_jw, preferred_element_type=jnp.float32))
        v1 = v1_ref[...]
        inv1 = g1_ref[...] * jax.lax.rsqrt(
            jnp.sum(v1 * v1, axis=1, keepdims=True))
        w1_ref[...] = v1 * inv1

    p = jnp.clip(xyzt_ref[...], 0.0, 1.0) * (L - 1)
    offs = jax.lax.broadcasted_iota(jnp.int32, (L, 1), 0).astype(jnp.float32)
    tent = jnp.concatenate(
        [jnp.maximum(1.0 - jnp.abs(p[a:a + 1, :] - offs), 0.0)
         for a in range(3)], axis=0)

    h = jnp.dot(mcat_ref[...], tent, preferred_element_type=jnp.float32)
    h = jnp.maximum(h + b0_ref[...], 0.0)
    h = jnp.dot(w1_ref[...], h, preferred_element_type=jnp.float32)
    h = jnp.maximum(h + b1_ref[...], 0.0)

    v2 = v2_ref[...]
    w2 = v2 * (g2_ref[...] * jax.lax.rsqrt(jnp.sum(v2 * v2)))
    out_ref[...] = (jnp.dot(w2, h, preferred_element_type=jnp.float32)
                    + b2_ref[...])


@jax.jit
def kernel(expr, jaw_quat_weight, xyz, feat_lines_x, feat_lines_y,
           feat_lines_z, v0, g0, b0, v1, g1, b1, v2, g2, b2):
    e = expr.reshape(-1)[:EXPR_NUM]
    jw = jaw_quat_weight.reshape(-1)
    nb = EXPR_NUM + KEY_JAW
    u_bs = jnp.concatenate([e, jnp.zeros((KEY_JAW,), e.dtype)])[:, None]
    u_jw = jnp.concatenate([jnp.zeros((EXPR_NUM,), jw.dtype), jw])[:, None]

    n = xyz.shape[0]
    xyzt = xyz.T
    flxt = jnp.swapaxes(feat_lines_x, 1, 2)
    flyt = jnp.swapaxes(feat_lines_y, 1, 2)
    flzt = jnp.swapaxes(feat_lines_z, 1, 2)

    grid = (n // BLOCK,)
    const = lambda shape: pl.BlockSpec(shape, lambda i: (0,) * len(shape))

    out = pl.pallas_call(
        _fused_kernel,
        grid=grid,
        in_specs=[
            pl.BlockSpec((3, BLOCK), lambda i: (0, i)),
            const((nb, 1)), const((nb, 1)),
            const((nb, C, L)), const((nb, C, L)), const((nb, C, L)),
            const((N_HIDDEN, 6 * C)), const((N_HIDDEN, 1)), const((N_HIDDEN, 1)),
            const((N_HIDDEN, N_HIDDEN)), const((N_HIDDEN, 1)), const((N_HIDDEN, 1)),
            const((1, N_HIDDEN)), const((1, 1)), const((1, 1)),
        ],
        out_specs=pl.BlockSpec((1, BLOCK), lambda i: (0, i)),
        out_shape=jax.ShapeDtypeStruct((1, n), jnp.float32),
        scratch_shapes=[
            pltpu.VMEM((N_HIDDEN, 3 * L), jnp.float32),
            pltpu.VMEM((N_HIDDEN, N_HIDDEN), jnp.float32),
        ],
    )(xyzt, u_bs, u_jw, flxt, flyt, flzt,
      v0, g0[:, None], b0[:, None], v1, g1[:, None], b1[:, None],
      v2, g2[:, None], b2[None, :])
    return out.reshape(n, 1)


# speedup vs baseline: 88.5678x; 1.0472x over previous
import jax
import jax.numpy as jnp
from jax.experimental import pallas as pl
from jax.experimental.pallas import tpu as pltpu

EXPR_NUM = 80
KEY_JAW = 16
L = 64
C = 32
N_HIDDEN = 128

BLOCK = 8192


def _fused_kernel(xyzt_ref, u_bs_ref, u_jw_ref, flx_ref, fly_ref, flz_ref,
                  v0_ref, g0_ref, b0_ref, v1_ref, g1_ref, b1_ref,
                  v2_ref, g2_ref, b2_ref,
                  out_ref,
                  mcat_ref, w1_ref):
    @pl.when(pl.program_id(0) == 0)
    def _prologue():
        v0 = v0_ref[...]
        inv0 = g0_ref[...] * jax.lax.rsqrt(
            jnp.sum(v0 * v0, axis=1, keepdims=True))
        w0 = v0 * inv0
        u_bs = u_bs_ref[...][:, :, None]
        u_jw = u_jw_ref[...][:, :, None]
        for a, fl_ref in ((0, flx_ref), (1, fly_ref), (2, flz_ref)):
            fl = fl_ref[...]
            a_bs = jnp.sum(fl * u_bs, axis=0)
            a_jw = jnp.sum(fl * u_jw, axis=0)
            w0_bs = w0[:, 32 * a:32 * (a + 1)]
            w0_jw = w0[:, 96 + 32 * a:96 + 32 * (a + 1)]
            mcat_ref[:, 64 * a:64 * (a + 1)] = (
                jnp.dot(w0_bs, a_bs, preferred_element_type=jnp.float32)
                + jnp.dot(w0_jw, a_jw, preferred_element_type=jnp.float32))
        v1 = v1_ref[...]
        inv1 = g1_ref[...] * jax.lax.rsqrt(
            jnp.sum(v1 * v1, axis=1, keepdims=True))
        w1_ref[...] = v1 * inv1

    p = jnp.clip(xyzt_ref[...], 0.0, 1.0) * (L - 1)
    offs = jax.lax.broadcasted_iota(jnp.int32, (L, 1), 0).astype(jnp.float32)
    tent = jnp.concatenate(
        [jnp.maximum(1.0 - jnp.abs(p[a:a + 1, :] - offs), 0.0)
         for a in range(3)], axis=0)

    h = jnp.dot(mcat_ref[...], tent, preferred_element_type=jnp.float32)
    h = jnp.maximum(h + b0_ref[...], 0.0)
    h = jnp.dot(w1_ref[...], h, preferred_element_type=jnp.float32)
    h = jnp.maximum(h + b1_ref[...], 0.0)

    v2 = v2_ref[...]
    w2 = v2 * (g2_ref[...] * jax.lax.rsqrt(jnp.sum(v2 * v2)))
    out_ref[...] = (jnp.dot(w2, h, preferred_element_type=jnp.float32)
                    + b2_ref[...])


@jax.jit
def kernel(expr, jaw_quat_weight, xyz, feat_lines_x, feat_lines_y,
           feat_lines_z, v0, g0, b0, v1, g1, b1, v2, g2, b2):
    e = expr.reshape(-1)[:EXPR_NUM]
    jw = jaw_quat_weight.reshape(-1)
    nb = EXPR_NUM + KEY_JAW
    u_bs = jnp.concatenate([e, jnp.zeros((KEY_JAW,), e.dtype)])[:, None]
    u_jw = jnp.concatenate([jnp.zeros((EXPR_NUM,), jw.dtype), jw])[:, None]

    n = xyz.shape[0]
    xyzt = xyz.T
    flxt = jnp.swapaxes(feat_lines_x, 1, 2)
    flyt = jnp.swapaxes(feat_lines_y, 1, 2)
    flzt = jnp.swapaxes(feat_lines_z, 1, 2)

    grid = (n // BLOCK,)
    const = lambda shape: pl.BlockSpec(shape, lambda i: (0,) * len(shape))

    out = pl.pallas_call(
        _fused_kernel,
        grid=grid,
        in_specs=[
            pl.BlockSpec((3, BLOCK), lambda i: (0, i)),
            const((nb, 1)), const((nb, 1)),
            const((nb, C, L)), const((nb, C, L)), const((nb, C, L)),
            const((N_HIDDEN, 6 * C)), const((N_HIDDEN, 1)), const((N_HIDDEN, 1)),
            const((N_HIDDEN, N_HIDDEN)), const((N_HIDDEN, 1)), const((N_HIDDEN, 1)),
            const((1, N_HIDDEN)), const((1, 1)), const((1, 1)),
        ],
        out_specs=pl.BlockSpec((1, BLOCK), lambda i: (0, i)),
        out_shape=jax.ShapeDtypeStruct((1, n), jnp.float32),
        scratch_shapes=[
            pltpu.VMEM((N_HIDDEN, 3 * L), jnp.float32),
            pltpu.VMEM((N_HIDDEN, N_HIDDEN), jnp.float32),
        ],
    )(xyzt, u_bs, u_jw, flxt, flyt, flzt,
      v0, g0[:, None], b0[:, None], v1, g1[:, None], b1[:, None],
      v2, g2[:, None], b2[None, :])
    return out.reshape(n, 1)
